# baseline (speedup 1.0000x reference)
import jax
import jax.numpy as jnp
from jax import lax
from jax.experimental import pallas as pl
from jax.experimental.pallas import tpu as pltpu
from jax.experimental.pallas import tpu_sc as plsc

N_NODES = 10000
N_EDGES = 320000
D_FEAT = 128
D_OUT = 128

NC = 2
NS = 16
NW = NC * NS

N_PAD = 10240
STRIPE = N_PAD // NS

EPW = N_EDGES // NW
CH = 80
NCH = EPW // CH


def _sc_body(ei_hbm, w_hbm, x_hbm, np_hbm, deg_hbm,
             src_all, w_all, src_ch, dst_ch, rows, dtmp,
             acc_sh, deg_sh, gsem0, gsem1, dsem0, dsem1, dsem2, dsem3,
             bsem0, bsem1, bsem2, bsem3, ssem0, ssem1):
  ssems = (ssem0, ssem1)
  degsems = (dsem0, dsem1, dsem2, dsem3)
  bsems = (bsem0, bsem1, bsem2, bsem3)
  c = lax.axis_index("c")
  s = lax.axis_index("s")
  wid = s * NC + c

  def _zrow(r, _):
    for j in range(D_FEAT // 16):
      rows[r, pl.ds(16 * j, 16)] = jnp.zeros((16,), jnp.float32)
    return 0
  lax.fori_loop(0, 2 * CH, _zrow, 0)
  for k in range(CH // 16):
    dtmp[pl.ds(16 * k, 16)] = jnp.zeros((16,), jnp.float32)
  for k in range(STRIPE // CH):
    r0 = s * STRIPE + k * CH
    pltpu.sync_copy(rows.at[pl.ds(0, CH)], acc_sh.at[pl.ds(r0, CH)])
    pltpu.sync_copy(dtmp, deg_sh.at[pl.ds(r0, CH)])
  plsc.subcore_barrier()

  base = wid * EPW
  pltpu.sync_copy(ei_hbm.at[pl.ds(base, EPW)], src_all)
  pltpu.sync_copy(w_hbm.at[pl.ds(base, EPW)], w_all)

  def idx_copy(ci, P):
    off = ci * CH
    for k in range(CH // 16):
      src_ch[P, pl.ds(16 * k, 16)] = src_all[pl.ds(off + 16 * k, 16)]
    pltpu.async_copy(
        ei_hbm.at[pl.ds(N_EDGES + base + off, CH)], dst_ch.at[P], bsems[P])

  def dst_wait(P):
    pltpu.make_async_copy(
        ei_hbm.at[pl.ds(0, CH)], dst_ch.at[P], bsems[P]).wait()

  def gather_start(P, R, sem):
    pltpu.async_copy(x_hbm.at[src_ch.at[P]], rows.at[pl.ds(R * CH, CH)], sem)

  def gather_wait(P, R, sem):
    pltpu.make_async_copy(
        x_hbm.at[src_ch.at[P]], rows.at[pl.ds(R * CH, CH)], sem).wait()

  def deg_wait(P, sem):
    pltpu.make_async_copy(dtmp, deg_sh.at[dst_ch.at[P]], sem).wait()

  def rows_scatter_wait(R):
    pltpu.make_async_copy(
        rows.at[pl.ds(R * CH, CH)], acc_sh.at[dst_ch.at[0]], ssems[R]).wait()

  def do_chunk(ci, P, R, gsem_, dsem_):
    gather_wait(P, R, gsem_)

    @plsc.parallel_loop(0, CH // 16)
    def _scale(k):
      e0 = 16 * k
      w16 = w_all[pl.ds(ci * CH + e0, 16)]
      for l in range(16):
        wv = w16[l]
        for j in range(D_FEAT // 16):
          sl = pl.ds(16 * j, 16)
          rows[R * CH + e0 + l, sl] = rows[R * CH + e0 + l, sl] * wv

    dst_wait(P)
    pltpu.async_copy(
        rows.at[pl.ds(R * CH, CH)], acc_sh.at[dst_ch.at[P]], ssems[R],
        add=True)
    pltpu.async_copy(
        w_all.at[pl.ds(ci * CH, CH)], deg_sh.at[dst_ch.at[P]], dsem_,
        add=True)

  for P in range(4):
    for k in range(CH // 16):
      dst_ch[P, pl.ds(16 * k, 16)] = jnp.zeros((16,), jnp.int32)
  for P in (1, 2, 3):
    pltpu.async_copy(dtmp, deg_sh.at[dst_ch.at[P]], degsems[P], add=True)
  pltpu.async_copy(
      rows.at[pl.ds(CH, CH)], acc_sh.at[dst_ch.at[0]], ssems[1], add=True)
  idx_copy(0, 0)
  gather_start(0, 0, gsem0)

  def _body(j, _):
    c0 = 4 * j
    deg_wait(1, degsems[1]); idx_copy(c0 + 1, 1)
    rows_scatter_wait(1); gather_start(1, 1, gsem1)
    do_chunk(c0, 0, 0, gsem0, degsems[0])
    deg_wait(2, degsems[2]); idx_copy(c0 + 2, 2)
    rows_scatter_wait(0); gather_start(2, 0, gsem0)
    do_chunk(c0 + 1, 1, 1, gsem1, degsems[1])
    deg_wait(3, degsems[3]); idx_copy(c0 + 3, 3)
    rows_scatter_wait(1); gather_start(3, 1, gsem1)
    do_chunk(c0 + 2, 2, 0, gsem0, degsems[2])
    deg_wait(0, degsems[0]); idx_copy(c0 + 4, 0)
    rows_scatter_wait(0); gather_start(0, 0, gsem0)
    do_chunk(c0 + 3, 3, 1, gsem1, degsems[3])
    return 0
  lax.fori_loop(0, (NCH - 1) // 4, _body, 0)
  do_chunk(NCH - 1, 0, 0, gsem0, degsems[0])
  for P in range(4):
    deg_wait(P, degsems[P])
  rows_scatter_wait(0)
  rows_scatter_wait(1)
  plsc.subcore_barrier()

  for k in range(STRIPE // CH):
    r0 = s * STRIPE + k * CH
    pltpu.sync_copy(acc_sh.at[pl.ds(r0, CH)], rows.at[pl.ds(0, CH)])
    pltpu.sync_copy(rows.at[pl.ds(0, CH)], np_hbm.at[c, pl.ds(r0, CH)])
    pltpu.sync_copy(deg_sh.at[pl.ds(r0, CH)], dtmp)
    pltpu.sync_copy(dtmp, deg_hbm.at[pl.ds(c * N_PAD + r0, CH)])


_sc_call = pl.kernel(
    _sc_body,
    out_type=(
        jax.ShapeDtypeStruct((NC, N_PAD, D_FEAT), jnp.float32),
        jax.ShapeDtypeStruct((NC * N_PAD,), jnp.float32),
    ),
    mesh=plsc.VectorSubcoreMesh(
        core_axis_name="c", subcore_axis_name="s", num_cores=NC,
        num_subcores=NS),
    scratch_types=(
        pltpu.VMEM((EPW,), jnp.int32),
        pltpu.VMEM((EPW,), jnp.float32),
        pltpu.VMEM((4, CH), jnp.int32),
        pltpu.VMEM((4, CH), jnp.int32),
        pltpu.VMEM((2 * CH, D_FEAT), jnp.float32),
        pltpu.VMEM((CH,), jnp.float32),
        pltpu.VMEM_SHARED((N_PAD, D_FEAT), jnp.float32),
        pltpu.VMEM_SHARED((N_PAD,), jnp.float32),
        pltpu.SemaphoreType.DMA,
        pltpu.SemaphoreType.DMA,
        pltpu.SemaphoreType.DMA,
        pltpu.SemaphoreType.DMA,
        pltpu.SemaphoreType.DMA,
        pltpu.SemaphoreType.DMA,
        pltpu.SemaphoreType.DMA,
        pltpu.SemaphoreType.DMA,
        pltpu.SemaphoreType.DMA,
        pltpu.SemaphoreType.DMA,
        pltpu.SemaphoreType.DMA,
        pltpu.SemaphoreType.DMA,
    ),
)


_TC_R = 1000


def _tc_body(x_ref, p0_ref, p1_ref, d0_ref, d1_ref, w1_ref, w2_ref, o_ref):
  d = d0_ref[...] + d1_ref[...]
  neigh = (p0_ref[0] + p1_ref[0]) / (d + 1e-9)
  acc = jnp.dot(x_ref[...], w1_ref[...], preferred_element_type=jnp.float32)
  acc = acc + jnp.dot(neigh, w2_ref[...], preferred_element_type=jnp.float32)
  o_ref[...] = acc * jax.nn.sigmoid(acc)


_tc_call = pl.pallas_call(
    _tc_body,
    grid=(N_NODES // _TC_R,),
    in_specs=[
        pl.BlockSpec((_TC_R, D_FEAT), lambda i: (i, 0)),
        pl.BlockSpec((1, _TC_R, D_FEAT), lambda i: (0, i, 0)),
        pl.BlockSpec((1, _TC_R, D_FEAT), lambda i: (1, i, 0)),
        pl.BlockSpec((_TC_R, 1), lambda i: (i, 0)),
        pl.BlockSpec((_TC_R, 1), lambda i: (i, 0)),
        pl.BlockSpec((D_FEAT, D_OUT), lambda i: (0, 0)),
        pl.BlockSpec((D_FEAT, D_OUT), lambda i: (0, 0)),
    ],
    out_specs=pl.BlockSpec((_TC_R, D_OUT), lambda i: (i, 0)),
    out_shape=jax.ShapeDtypeStruct((N_NODES, D_OUT), jnp.float32),
)


@jax.jit
def kernel(x, edge_index, edge_weight, W):
  ei = edge_index.astype(jnp.int32).reshape(2 * N_EDGES)
  w = edge_weight.astype(jnp.float32)
  np_out, deg_out = _sc_call(ei, w, x)
  d0 = deg_out[:N_NODES].reshape(N_NODES, 1)
  d1 = deg_out[N_PAD:N_PAD + N_NODES].reshape(N_NODES, 1)
  return _tc_call(x, np_out, np_out, d0, d1, W[:D_FEAT], W[D_FEAT:])


# speedup vs baseline: 1.1824x; 1.1824x over previous
import jax
import jax.numpy as jnp
from jax import lax
from jax.experimental import pallas as pl
from jax.experimental.pallas import tpu as pltpu
from jax.experimental.pallas import tpu_sc as plsc

N_NODES = 10000
N_EDGES = 320000
D_FEAT = 128
D_OUT = 128

NC = 2
NS = 16
NW = NC * NS

N_PAD = 10240
STRIPE = N_PAD // NS

EPW = N_EDGES // NW
CH = 80
NCH = EPW // CH


def _sc_body(ei_hbm, w_hbm, x_hbm, np_hbm, deg_hbm,
             src_all, w_all, src_ch, dst_ch, rows, dtmp,
             acc_sh, deg_sh, gsem0, gsem1, dsem0, dsem1, dsem2, dsem3,
             bsem0, bsem1, bsem2, bsem3):
  degsems = (dsem0, dsem1, dsem2, dsem3)
  bsems = (bsem0, bsem1, bsem2, bsem3)
  c = lax.axis_index("c")
  s = lax.axis_index("s")
  wid = s * NC + c

  def _zrow(r, _):
    for j in range(D_FEAT // 16):
      rows[r, pl.ds(16 * j, 16)] = jnp.zeros((16,), jnp.float32)
    return 0
  lax.fori_loop(0, 2 * CH, _zrow, 0)
  for k in range(CH // 16):
    dtmp[pl.ds(16 * k, 16)] = jnp.zeros((16,), jnp.float32)
  for k in range(STRIPE // CH):
    r0 = s * STRIPE + k * CH
    pltpu.sync_copy(rows.at[pl.ds(0, CH)], acc_sh.at[pl.ds(r0, CH)])
    pltpu.sync_copy(dtmp, deg_sh.at[pl.ds(r0, CH)])
  plsc.subcore_barrier()

  base = wid * EPW
  pltpu.sync_copy(ei_hbm.at[pl.ds(base, EPW)], src_all)
  pltpu.sync_copy(w_hbm.at[pl.ds(base, EPW)], w_all)

  def idx_copy(ci, P):
    off = ci * CH
    for k in range(CH // 16):
      src_ch[P, pl.ds(16 * k, 16)] = src_all[pl.ds(off + 16 * k, 16)]
    pltpu.async_copy(
        ei_hbm.at[pl.ds(N_EDGES + base + off, CH)], dst_ch.at[P], bsems[P])

  def dst_wait(P):
    pltpu.make_async_copy(
        ei_hbm.at[pl.ds(0, CH)], dst_ch.at[P], bsems[P]).wait()

  def gather_start(P, R, sem):
    pltpu.async_copy(x_hbm.at[src_ch.at[P]], rows.at[pl.ds(R * CH, CH)], sem)

  def gather_wait(P, R, sem):
    pltpu.make_async_copy(
        x_hbm.at[src_ch.at[P]], rows.at[pl.ds(R * CH, CH)], sem).wait()

  def deg_wait(P, sem):
    pltpu.make_async_copy(dtmp, deg_sh.at[dst_ch.at[P]], sem).wait()

  def do_chunk(ci, P, R, gsem_, dsem_):
    gather_wait(P, R, gsem_)

    def _scale(k, _):
      e0 = 16 * k
      w16 = w_all[pl.ds(ci * CH + e0, 16)]
      for l in range(16):
        wv = w16[l]
        for j in range(D_FEAT // 16):
          sl = pl.ds(16 * j, 16)
          rows[R * CH + e0 + l, sl] = rows[R * CH + e0 + l, sl] * wv
      return 0
    lax.fori_loop(0, CH // 16, _scale, 0)

    dst_wait(P)
    pltpu.sync_copy(
        rows.at[pl.ds(R * CH, CH)], acc_sh.at[dst_ch.at[P]], add=True)
    pltpu.async_copy(
        w_all.at[pl.ds(ci * CH, CH)], deg_sh.at[dst_ch.at[P]], dsem_,
        add=True)

  for P in range(4):
    for k in range(CH // 16):
      dst_ch[P, pl.ds(16 * k, 16)] = jnp.zeros((16,), jnp.int32)
  for P in (1, 2, 3):
    pltpu.async_copy(dtmp, deg_sh.at[dst_ch.at[P]], degsems[P], add=True)
  idx_copy(0, 0)
  gather_start(0, 0, gsem0)

  def _body(j, _):
    c0 = 4 * j
    deg_wait(1, degsems[1]); idx_copy(c0 + 1, 1); gather_start(1, 1, gsem1)
    do_chunk(c0, 0, 0, gsem0, degsems[0])
    deg_wait(2, degsems[2]); idx_copy(c0 + 2, 2); gather_start(2, 0, gsem0)
    do_chunk(c0 + 1, 1, 1, gsem1, degsems[1])
    deg_wait(3, degsems[3]); idx_copy(c0 + 3, 3); gather_start(3, 1, gsem1)
    do_chunk(c0 + 2, 2, 0, gsem0, degsems[2])
    deg_wait(0, degsems[0]); idx_copy(c0 + 4, 0); gather_start(0, 0, gsem0)
    do_chunk(c0 + 3, 3, 1, gsem1, degsems[3])
    return 0
  lax.fori_loop(0, (NCH - 1) // 4, _body, 0)
  do_chunk(NCH - 1, 0, 0, gsem0, degsems[0])
  for P in range(4):
    deg_wait(P, degsems[P])
  plsc.subcore_barrier()

  for k in range(STRIPE // CH):
    r0 = s * STRIPE + k * CH
    pltpu.sync_copy(acc_sh.at[pl.ds(r0, CH)], rows.at[pl.ds(0, CH)])
    pltpu.sync_copy(rows.at[pl.ds(0, CH)], np_hbm.at[c, pl.ds(r0, CH)])
    pltpu.sync_copy(deg_sh.at[pl.ds(r0, CH)], dtmp)
    pltpu.sync_copy(dtmp, deg_hbm.at[pl.ds(c * N_PAD + r0, CH)])


_sc_call = pl.kernel(
    _sc_body,
    out_type=(
        jax.ShapeDtypeStruct((NC, N_PAD, D_FEAT), jnp.float32),
        jax.ShapeDtypeStruct((NC * N_PAD,), jnp.float32),
    ),
    mesh=plsc.VectorSubcoreMesh(
        core_axis_name="c", subcore_axis_name="s", num_cores=NC,
        num_subcores=NS),
    scratch_types=(
        pltpu.VMEM((EPW,), jnp.int32),
        pltpu.VMEM((EPW,), jnp.float32),
        pltpu.VMEM((4, CH), jnp.int32),
        pltpu.VMEM((4, CH), jnp.int32),
        pltpu.VMEM((2 * CH, D_FEAT), jnp.float32),
        pltpu.VMEM((CH,), jnp.float32),
        pltpu.VMEM_SHARED((N_PAD, D_FEAT), jnp.float32),
        pltpu.VMEM_SHARED((N_PAD,), jnp.float32),
        pltpu.SemaphoreType.DMA,
        pltpu.SemaphoreType.DMA,
        pltpu.SemaphoreType.DMA,
        pltpu.SemaphoreType.DMA,
        pltpu.SemaphoreType.DMA,
        pltpu.SemaphoreType.DMA,
        pltpu.SemaphoreType.DMA,
        pltpu.SemaphoreType.DMA,
        pltpu.SemaphoreType.DMA,
        pltpu.SemaphoreType.DMA,
    ),
)


_TC_R = 1000


def _tc_body(x_ref, p0_ref, p1_ref, d0_ref, d1_ref, w1_ref, w2_ref, o_ref):
  d = d0_ref[...] + d1_ref[...]
  neigh = (p0_ref[0] + p1_ref[0]) / (d + 1e-9)
  acc = jnp.dot(x_ref[...], w1_ref[...], preferred_element_type=jnp.float32)
  acc = acc + jnp.dot(neigh, w2_ref[...], preferred_element_type=jnp.float32)
  o_ref[...] = acc * jax.nn.sigmoid(acc)


_tc_call = pl.pallas_call(
    _tc_body,
    grid=(N_NODES // _TC_R,),
    in_specs=[
        pl.BlockSpec((_TC_R, D_FEAT), lambda i: (i, 0)),
        pl.BlockSpec((1, _TC_R, D_FEAT), lambda i: (0, i, 0)),
        pl.BlockSpec((1, _TC_R, D_FEAT), lambda i: (1, i, 0)),
        pl.BlockSpec((_TC_R, 1), lambda i: (i, 0)),
        pl.BlockSpec((_TC_R, 1), lambda i: (i, 0)),
        pl.BlockSpec((D_FEAT, D_OUT), lambda i: (0, 0)),
        pl.BlockSpec((D_FEAT, D_OUT), lambda i: (0, 0)),
    ],
    out_specs=pl.BlockSpec((_TC_R, D_OUT), lambda i: (i, 0)),
    out_shape=jax.ShapeDtypeStruct((N_NODES, D_OUT), jnp.float32),
)


@jax.jit
def kernel(x, edge_index, edge_weight, W):
  ei = edge_index.astype(jnp.int32).reshape(2 * N_EDGES)
  w = edge_weight.astype(jnp.float32)
  np_out, deg_out = _sc_call(ei, w, x)
  d0 = deg_out[:N_NODES].reshape(N_NODES, 1)
  d1 = deg_out[N_PAD:N_PAD + N_NODES].reshape(N_NODES, 1)
  return _tc_call(x, np_out, np_out, d0, d1, W[:D_FEAT], W[D_FEAT:])


# speedup vs baseline: 1.2133x; 1.0262x over previous
import jax
import jax.numpy as jnp
from jax import lax
from jax.experimental import pallas as pl
from jax.experimental.pallas import tpu as pltpu
from jax.experimental.pallas import tpu_sc as plsc

N_NODES = 10000
N_EDGES = 320000
D_FEAT = 128
D_OUT = 128

NC = 2
NS = 16
NW = NC * NS

N_PAD = 10240
STRIPE = N_PAD // NS

EPW = N_EDGES // NW
CH = 80
NCH = EPW // CH


def _sc_body(ei_hbm, w_hbm, x_hbm, np_hbm, deg_hbm,
             src_all, w_all, src_ch, dst_ch, rows, dtmp,
             acc_sh, deg_sh, gsem0, gsem1, dsem0, dsem1, dsem2, dsem3,
             bsem0, bsem1, bsem2, bsem3):
  degsems = (dsem0, dsem1, dsem2, dsem3)
  bsems = (bsem0, bsem1, bsem2, bsem3)
  c = lax.axis_index("c")
  s = lax.axis_index("s")
  wid = s * NC + c

  def _zrow(r, _):
    for j in range(D_FEAT // 16):
      rows[r, pl.ds(16 * j, 16)] = jnp.zeros((16,), jnp.float32)
    return 0
  lax.fori_loop(0, 2 * CH, _zrow, 0)
  for k in range(CH // 16):
    dtmp[pl.ds(16 * k, 16)] = jnp.zeros((16,), jnp.float32)
  for k in range(STRIPE // CH):
    r0 = s * STRIPE + k * CH
    pltpu.sync_copy(rows.at[pl.ds(0, CH)], acc_sh.at[pl.ds(r0, CH)])
    pltpu.sync_copy(dtmp, deg_sh.at[pl.ds(r0, CH)])
  plsc.subcore_barrier()

  base = wid * EPW
  pltpu.sync_copy(ei_hbm.at[pl.ds(base, EPW)], src_all)
  pltpu.sync_copy(w_hbm.at[pl.ds(base, EPW)], w_all)

  def idx_copy(ci, P):
    off = ci * CH
    for k in range(CH // 16):
      src_ch[P, pl.ds(16 * k, 16)] = src_all[pl.ds(off + 16 * k, 16)]
    pltpu.async_copy(
        ei_hbm.at[pl.ds(N_EDGES + base + off, CH)], dst_ch.at[P], bsems[P])

  def dst_wait(P):
    pltpu.make_async_copy(
        ei_hbm.at[pl.ds(0, CH)], dst_ch.at[P], bsems[P]).wait()

  def gather_start(P, R, sem):
    pltpu.async_copy(x_hbm.at[src_ch.at[P]], rows.at[pl.ds(R * CH, CH)], sem)

  def gather_wait(P, R, sem):
    pltpu.make_async_copy(
        x_hbm.at[src_ch.at[P]], rows.at[pl.ds(R * CH, CH)], sem).wait()

  def deg_wait(P, sem):
    pltpu.make_async_copy(dtmp, deg_sh.at[dst_ch.at[P]], sem).wait()

  def do_chunk(ci, P, R, gsem_, dsem_):
    gather_wait(P, R, gsem_)

    def _scale(k, _):
      e0 = 16 * k
      w16 = w_all[pl.ds(ci * CH + e0, 16)]
      for l in range(16):
        wv = w16[l]
        for j in range(D_FEAT // 16):
          sl = pl.ds(16 * j, 16)
          rows[R * CH + e0 + l, sl] = rows[R * CH + e0 + l, sl] * wv
      return 0
    lax.fori_loop(0, CH // 16, _scale, 0)

    dst_wait(P)
    pltpu.sync_copy(
        rows.at[pl.ds(R * CH, CH)], acc_sh.at[dst_ch.at[P]], add=True)
    pltpu.async_copy(
        w_all.at[pl.ds(ci * CH, CH)], deg_sh.at[dst_ch.at[P]], dsem_,
        add=True)

  for P in range(4):
    for k in range(CH // 16):
      dst_ch[P, pl.ds(16 * k, 16)] = jnp.zeros((16,), jnp.int32)
  for P in (1, 2, 3):
    pltpu.async_copy(dtmp, deg_sh.at[dst_ch.at[P]], degsems[P], add=True)
  idx_copy(0, 0)
  gather_start(0, 0, gsem0)

  def _body(j, _):
    c0 = 4 * j
    deg_wait(1, degsems[1]); idx_copy(c0 + 1, 1); gather_start(1, 1, gsem1)
    do_chunk(c0, 0, 0, gsem0, degsems[0])
    deg_wait(2, degsems[2]); idx_copy(c0 + 2, 2); gather_start(2, 0, gsem0)
    do_chunk(c0 + 1, 1, 1, gsem1, degsems[1])
    deg_wait(3, degsems[3]); idx_copy(c0 + 3, 3); gather_start(3, 1, gsem1)
    do_chunk(c0 + 2, 2, 0, gsem0, degsems[2])
    deg_wait(0, degsems[0]); idx_copy(c0 + 4, 0); gather_start(0, 0, gsem0)
    do_chunk(c0 + 3, 3, 1, gsem1, degsems[3])
    return 0
  lax.fori_loop(0, (NCH - 1) // 4, _body, 0)
  do_chunk(NCH - 1, 0, 0, gsem0, degsems[0])
  for P in range(4):
    deg_wait(P, degsems[P])
  plsc.subcore_barrier()

  for k in range(STRIPE // CH):
    r0 = s * STRIPE + k * CH
    R = k % 2
    if k >= 2:
      r0p = s * STRIPE + (k - 2) * CH
      pltpu.make_async_copy(
          rows.at[pl.ds(R * CH, CH)], np_hbm.at[c, pl.ds(r0p, CH)],
          (gsem0, gsem1)[R]).wait()
    pltpu.sync_copy(acc_sh.at[pl.ds(r0, CH)], rows.at[pl.ds(R * CH, CH)])
    pltpu.async_copy(rows.at[pl.ds(R * CH, CH)], np_hbm.at[c, pl.ds(r0, CH)],
                     (gsem0, gsem1)[R])
    pltpu.sync_copy(deg_sh.at[pl.ds(r0, CH)], dtmp)
    pltpu.sync_copy(dtmp, deg_hbm.at[pl.ds(c * N_PAD + r0, CH)])
  for R in range(2):
    r0p = s * STRIPE + (STRIPE // CH - 2 + R) * CH
    pltpu.make_async_copy(
        rows.at[pl.ds(R * CH, CH)], np_hbm.at[c, pl.ds(r0p, CH)],
        (gsem0, gsem1)[R]).wait()


_sc_call = pl.kernel(
    _sc_body,
    out_type=(
        jax.ShapeDtypeStruct((NC, N_PAD, D_FEAT), jnp.float32),
        jax.ShapeDtypeStruct((NC * N_PAD,), jnp.float32),
    ),
    mesh=plsc.VectorSubcoreMesh(
        core_axis_name="c", subcore_axis_name="s", num_cores=NC,
        num_subcores=NS),
    scratch_types=(
        pltpu.VMEM((EPW,), jnp.int32),
        pltpu.VMEM((EPW,), jnp.float32),
        pltpu.VMEM((4, CH), jnp.int32),
        pltpu.VMEM((4, CH), jnp.int32),
        pltpu.VMEM((2 * CH, D_FEAT), jnp.float32),
        pltpu.VMEM((CH,), jnp.float32),
        pltpu.VMEM_SHARED((N_PAD, D_FEAT), jnp.float32),
        pltpu.VMEM_SHARED((N_PAD,), jnp.float32),
        pltpu.SemaphoreType.DMA,
        pltpu.SemaphoreType.DMA,
        pltpu.SemaphoreType.DMA,
        pltpu.SemaphoreType.DMA,
        pltpu.SemaphoreType.DMA,
        pltpu.SemaphoreType.DMA,
        pltpu.SemaphoreType.DMA,
        pltpu.SemaphoreType.DMA,
        pltpu.SemaphoreType.DMA,
        pltpu.SemaphoreType.DMA,
    ),
)


_TC_R = 2000


def _tc_body(x_ref, p0_ref, p1_ref, d0_ref, d1_ref, w1_ref, w2_ref, o_ref):
  d = d0_ref[...] + d1_ref[...]
  neigh = (p0_ref[0] + p1_ref[0]) / (d + 1e-9)
  acc = jnp.dot(x_ref[...], w1_ref[...], preferred_element_type=jnp.float32)
  acc = acc + jnp.dot(neigh, w2_ref[...], preferred_element_type=jnp.float32)
  o_ref[...] = acc * jax.nn.sigmoid(acc)


_tc_call = pl.pallas_call(
    _tc_body,
    grid=(N_NODES // _TC_R,),
    in_specs=[
        pl.BlockSpec((_TC_R, D_FEAT), lambda i: (i, 0)),
        pl.BlockSpec((1, _TC_R, D_FEAT), lambda i: (0, i, 0)),
        pl.BlockSpec((1, _TC_R, D_FEAT), lambda i: (1, i, 0)),
        pl.BlockSpec((_TC_R, 1), lambda i: (i, 0)),
        pl.BlockSpec((_TC_R, 1), lambda i: (i, 0)),
        pl.BlockSpec((D_FEAT, D_OUT), lambda i: (0, 0)),
        pl.BlockSpec((D_FEAT, D_OUT), lambda i: (0, 0)),
    ],
    out_specs=pl.BlockSpec((_TC_R, D_OUT), lambda i: (i, 0)),
    out_shape=jax.ShapeDtypeStruct((N_NODES, D_OUT), jnp.float32),
)


@jax.jit
def kernel(x, edge_index, edge_weight, W):
  ei = edge_index.astype(jnp.int32).reshape(2 * N_EDGES)
  w = edge_weight.astype(jnp.float32)
  np_out, deg_out = _sc_call(ei, w, x)
  d0 = deg_out[:N_NODES].reshape(N_NODES, 1)
  d1 = deg_out[N_PAD:N_PAD + N_NODES].reshape(N_NODES, 1)
  return _tc_call(x, np_out, np_out, d0, d1, W[:D_FEAT], W[D_FEAT:])
